# Initial kernel scaffold; baseline (speedup 1.0000x reference)
import jax
import jax.numpy as jnp
from jax.experimental import pallas as pl


def kernel(feature, adj, members, nonmembers, leaders, weight, bias):
    raise NotImplementedError("write your pallas kernel here")


# speedup vs baseline: 1.3459x; 1.3459x over previous
import functools

import jax
import jax.numpy as jnp
from jax import lax
from jax.experimental import pallas as pl
from jax.experimental.pallas import tpu as pltpu
from jax.experimental.pallas import tpu_sc as plsc

N = 4096
ROLE = 1024
OUT_CH = 64
NROWS = 28
BN = 512
BK = 512
NI = N // BN
NK = N // BK


def _sc_counts_body(idx_hbm, out_hbm, idx_v, acc_v):
    cid = lax.axis_index("c")
    sid = lax.axis_index("s")
    wid = sid * 2 + cid

    @pl.when(wid < 3)
    def _():
        pltpu.sync_copy(idx_hbm.at[wid], idx_v)

        zeros16 = jnp.zeros((16,), jnp.float32)

        def zero_body(j, carry):
            acc_v[pl.ds(j * 16, 16)] = zeros16
            return carry

        lax.fori_loop(0, N // 16, zero_body, 0)

        ones16 = jnp.ones((16,), jnp.float32)

        def scat_body(j, carry):
            iv = idx_v[pl.ds(j * 16, 16)]
            plsc.addupdate_scatter(acc_v, [iv], ones16)
            return carry

        lax.fori_loop(0, ROLE // 16, scat_body, 0)

        pltpu.sync_copy(acc_v, out_hbm.at[wid])


def _sc_counts(idx_all):
    return pl.kernel(
        _sc_counts_body,
        out_type=jax.ShapeDtypeStruct((3, N), jnp.float32),
        mesh=plsc.VectorSubcoreMesh(core_axis_name="c", subcore_axis_name="s"),
        scratch_types=[
            pltpu.VMEM((ROLE,), jnp.int32),
            pltpu.VMEM((N,), jnp.float32),
        ],
        compiler_params=pltpu.CompilerParams(needs_layout_passes=False),
    )(idx_all)


def _tc_body(adj_ref, ct_ref, fk_ref, fi_ref, w2_ref, b_ref, out_ref, acc_ref):
    i = pl.program_id(0)
    k = pl.program_id(1)

    @pl.when(k == 0)
    def _():
        acc_ref[...] = jnp.zeros_like(acc_ref)

    adj = adj_ref[...]
    ct = ct_ref[...]
    f = fk_ref[...]
    g = jnp.concatenate(
        [ct[:, 0:1] * f, ct[:, 1:2] * f, ct[:, 2:3] * f], axis=1
    ) * (1.0 / ROLE)
    gb = g.astype(jnp.bfloat16)

    acc = acc_ref[...] + lax.dot_general(
        adj.astype(jnp.bfloat16), gb,
        (((1,), (0,)), ((), ())),
        preferred_element_type=jnp.float32,
    )

    @pl.when(k == i)
    def _():
        rows = lax.broadcasted_iota(jnp.int32, (BN, BK), 0)
        cols = lax.broadcasted_iota(jnp.int32, (BN, BK), 1)
        diag = jnp.sum(
            jnp.where(rows == cols, adj, 0.0), axis=1, keepdims=True
        )
        acc_ref[...] = acc - diag * gb.astype(jnp.float32)

    @pl.when(k != i)
    def _():
        acc_ref[...] = acc

    @pl.when(k == NK - 1)
    def _():
        h0 = jnp.concatenate([acc_ref[...], fi_ref[...]], axis=1)
        out = lax.dot_general(
            h0.astype(jnp.bfloat16), w2_ref[...],
            (((1,), (0,)), ((), ())),
            preferred_element_type=jnp.float32,
        )
        out_ref[...] = out + b_ref[...]


def _tc_call(adj, counts_t, f2, w2, btile):
    return pl.pallas_call(
        _tc_body,
        grid=(NI, NK),
        in_specs=[
            pl.BlockSpec((BN, BK), lambda i, k: (i, k)),
            pl.BlockSpec((BK, 3), lambda i, k: (k, 0)),
            pl.BlockSpec((BK, 7), lambda i, k: (k, 0)),
            pl.BlockSpec((BN, 7), lambda i, k: (i, 0)),
            pl.BlockSpec((NROWS, NROWS * OUT_CH), lambda i, k: (0, 0)),
            pl.BlockSpec((1, NROWS * OUT_CH), lambda i, k: (0, 0)),
        ],
        out_specs=pl.BlockSpec((BN, NROWS * OUT_CH), lambda i, k: (i, 0)),
        out_shape=jax.ShapeDtypeStruct((N, NROWS * OUT_CH), jnp.float32),
        scratch_shapes=[pltpu.VMEM((BN, 21), jnp.float32)],
    )(adj, counts_t, f2, f2, w2, btile)


def kernel(feature, adj, members, nonmembers, leaders, weight, bias):
    f2 = feature.reshape(N, 7)
    idx_all = jnp.stack([
        leaders.astype(jnp.int32),
        nonmembers.astype(jnp.int32),
        members.astype(jnp.int32),
    ])
    counts = _sc_counts(idx_all)
    counts_t = counts.T
    w2 = jnp.kron(jnp.eye(NROWS, dtype=weight.dtype), weight).astype(jnp.bfloat16)
    btile = jnp.tile(bias, NROWS).reshape(1, NROWS * OUT_CH)
    out2d = _tc_call(adj, counts_t, f2, w2, btile)
    return out2d.reshape(N, NROWS, OUT_CH)


# speedup vs baseline: 1.9533x; 1.4513x over previous
import functools

import jax
import jax.numpy as jnp
from jax import lax
from jax.experimental import pallas as pl
from jax.experimental.pallas import tpu as pltpu
from jax.experimental.pallas import tpu_sc as plsc

N = 4096
ROLE = 1024
OUT_CH = 64
NROWS = 28
BN = 512
NI = N // BN


def _sc_counts_body(idx_hbm, out_hbm, idx_v, acc_v):
    cid = lax.axis_index("c")
    sid = lax.axis_index("s")
    wid = sid * 2 + cid

    @pl.when(wid < 3)
    def _():
        pltpu.sync_copy(idx_hbm.at[wid], idx_v)

        zeros16 = jnp.zeros((16,), jnp.float32)

        def zero_body(j, carry):
            acc_v[pl.ds(j * 16, 16)] = zeros16
            return carry

        lax.fori_loop(0, N // 16, zero_body, 0)

        ones16 = jnp.ones((16,), jnp.float32)

        def scat_body(j, carry):
            iv = idx_v[pl.ds(j * 16, 16)]
            plsc.addupdate_scatter(acc_v, [iv], ones16)
            return carry

        lax.fori_loop(0, ROLE // 16, scat_body, 0)

        pltpu.sync_copy(acc_v, out_hbm.at[wid])


def _sc_counts(idx_all):
    return pl.kernel(
        _sc_counts_body,
        out_type=jax.ShapeDtypeStruct((3, N), jnp.float32),
        mesh=plsc.VectorSubcoreMesh(core_axis_name="c", subcore_axis_name="s"),
        scratch_types=[
            pltpu.VMEM((ROLE,), jnp.int32),
            pltpu.VMEM((N,), jnp.float32),
        ],
        compiler_params=pltpu.CompilerParams(needs_layout_passes=False),
    )(idx_all)


def _tc_body(adj_ref, ct_ref, f_ref, cti_ref, fi_ref, w2_ref, b_ref, out_ref,
             g_ref):
    i = pl.program_id(0)

    @pl.when(i == 0)
    def _():
        ct = ct_ref[...]
        f = f_ref[...]
        g = jnp.concatenate(
            [ct[:, 0:1] * f, ct[:, 1:2] * f, ct[:, 2:3] * f], axis=1
        ) * (1.0 / ROLE)
        g_ref[...] = g.astype(jnp.bfloat16)

    adj = adj_ref[...]
    gb = g_ref[...]

    acc = lax.dot_general(
        adj.astype(jnp.bfloat16), gb,
        (((1,), (0,)), ((), ())),
        preferred_element_type=jnp.float32,
    )

    rows = lax.broadcasted_iota(jnp.int32, (BN, N), 0) + i * BN
    cols = lax.broadcasted_iota(jnp.int32, (BN, N), 1)
    diag = jnp.sum(
        jnp.where(rows == cols, adj, 0.0), axis=1, keepdims=True
    )
    cti = cti_ref[...]
    fi = fi_ref[...]
    gi = jnp.concatenate(
        [cti[:, 0:1] * fi, cti[:, 1:2] * fi, cti[:, 2:3] * fi], axis=1
    ) * (1.0 / ROLE)
    h = acc - diag * gi

    h0 = jnp.concatenate([h, fi], axis=1)
    out = lax.dot_general(
        h0.astype(jnp.bfloat16), w2_ref[...],
        (((1,), (0,)), ((), ())),
        preferred_element_type=jnp.float32,
    )
    out_ref[...] = out + b_ref[...]


def _tc_call(adj, counts_t, f2, w2, btile):
    return pl.pallas_call(
        _tc_body,
        grid=(NI,),
        in_specs=[
            pl.BlockSpec((BN, N), lambda i: (i, 0)),
            pl.BlockSpec((N, 3), lambda i: (0, 0)),
            pl.BlockSpec((N, 7), lambda i: (0, 0)),
            pl.BlockSpec((BN, 3), lambda i: (i, 0)),
            pl.BlockSpec((BN, 7), lambda i: (i, 0)),
            pl.BlockSpec((NROWS, NROWS * OUT_CH), lambda i: (0, 0)),
            pl.BlockSpec((1, NROWS * OUT_CH), lambda i: (0, 0)),
        ],
        out_specs=pl.BlockSpec((BN, NROWS * OUT_CH), lambda i: (i, 0)),
        out_shape=jax.ShapeDtypeStruct((N, NROWS * OUT_CH), jnp.float32),
        scratch_shapes=[pltpu.VMEM((N, 21), jnp.bfloat16)],
    )(adj, counts_t, f2, counts_t, f2, w2, btile)


def kernel(feature, adj, members, nonmembers, leaders, weight, bias):
    f2 = feature.reshape(N, 7)
    idx_all = jnp.stack([
        leaders.astype(jnp.int32),
        nonmembers.astype(jnp.int32),
        members.astype(jnp.int32),
    ])
    counts = _sc_counts(idx_all)
    counts_t = counts.T
    w2 = jnp.kron(jnp.eye(NROWS, dtype=weight.dtype), weight).astype(jnp.bfloat16)
    btile = jnp.tile(bias, NROWS).reshape(1, NROWS * OUT_CH)
    out2d = _tc_call(adj, counts_t, f2, w2, btile)
    return out2d.reshape(N, NROWS, OUT_CH)
